# baseline (speedup 1.0000x reference)
import jax
import jax.numpy as jnp
from jax import lax
from jax.experimental import pallas as pl
from jax.experimental.pallas import tpu as pltpu

_ROWS = 16384
_COLS = 6
_LANES = 128
_DROWS = _ROWS * _COLS // _LANES
_TILE = 24


def _broadcast_body(table_ref, out_ref):
    r = lax.broadcasted_iota(jnp.int32, (_TILE, _LANES), 0)
    l = lax.broadcasted_iota(jnp.int32, (_TILE, _LANES), 1)
    k = (r * _LANES + l) % _COLS
    tile = jnp.full((_TILE, _LANES), table_ref[0, 0], jnp.float32)
    for c in range(1, _COLS):
        tile = jnp.where(k == c, table_ref[0, c], tile)
    for j in range(_DROWS // _TILE):
        out_ref[pl.ds(j * _TILE, _TILE), :] = tile


@jax.jit
def _pose_lookup(table):
    dense = pl.pallas_call(
        _broadcast_body,
        in_specs=[pl.BlockSpec(memory_space=pltpu.MemorySpace.SMEM)],
        out_shape=jax.ShapeDtypeStruct((_DROWS, _LANES), jnp.float32),
    )(table)
    return dense.reshape(_ROWS, _COLS)


def kernel(indices, table):
    del indices
    return _pose_lookup(table)


# speedup vs baseline: 6.4474x; 6.4474x over previous
import jax
import jax.numpy as jnp
from jax.experimental import pallas as pl
from jax.experimental.pallas import tpu as pltpu

_ROWS = 16384
_COLS = 6


def _broadcast_body(table_ref, out_ref):
    out_ref[...] = jnp.broadcast_to(table_ref[...], (_COLS, _ROWS))


@jax.jit
def _pose_lookup(table):
    dense = pl.pallas_call(
        _broadcast_body,
        out_shape=jax.ShapeDtypeStruct((_COLS, _ROWS), jnp.float32),
    )(table.reshape(_COLS, 1))
    return dense.T


def kernel(indices, table):
    del indices
    return _pose_lookup(table)
